# Initial kernel scaffold; baseline (speedup 1.0000x reference)
import jax
import jax.numpy as jnp
from jax.experimental import pallas as pl


def kernel(x, edge_index, edge_attr, params):
    raise NotImplementedError("write your pallas kernel here")


# speedup vs baseline: 6.7677x; 6.7677x over previous
import functools

import jax
import jax.numpy as jnp
from jax import lax
from jax.experimental import pallas as pl
from jax.experimental.pallas import tpu as pltpu
from jax.experimental.pallas import tpu_sc as plsc

F32 = jnp.float32

_BN = 400
_BE = 2000
_D = 128
_DH = 64
_C = 80
_NW = 32
_NT = 16


def _edge_feat_body(ea_ref, ve_ref, out_ref):
    out_ref[...] = jnp.dot(ea_ref[...], ve_ref[...], preferred_element_type=F32)


def _edge_feats(edge_attr, ve):
    E, DE = edge_attr.shape
    return pl.pallas_call(
        _edge_feat_body,
        grid=(E // _BE,),
        in_specs=[
            pl.BlockSpec((_BE, DE), lambda i: (i, 0)),
            pl.BlockSpec(ve.shape, lambda i: (0, 0)),
        ],
        out_specs=pl.BlockSpec((_BE, ve.shape[1]), lambda i: (i, 0)),
        out_shape=jax.ShapeDtypeStruct((E, ve.shape[1]), F32),
    )(edge_attr, ve)


def _scale_body(g_ref, a_ref, ma_ref, mb_ref):
    m = g_ref[...] * a_ref[...]
    ma_ref[...] = m[:, :_DH]
    mb_ref[...] = m[:, _DH:]


def _scale_rows(grows, a2d):
    E = grows.shape[0]
    return pl.pallas_call(
        _scale_body,
        grid=(E // _BE,),
        in_specs=[
            pl.BlockSpec((_BE, _D), lambda i: (i, 0)),
            pl.BlockSpec((_BE, 1), lambda i: (i, 0)),
        ],
        out_specs=[
            pl.BlockSpec((_BE, _DH), lambda i: (i, 0)),
            pl.BlockSpec((_BE, _DH), lambda i: (i, 0)),
        ],
        out_shape=[
            jax.ShapeDtypeStruct((E, _DH), F32),
            jax.ShapeDtypeStruct((E, _DH), F32),
        ],
    )(grows, a2d)


def _sd_from_xw(xw, a):
    s = jnp.sum(xw * a[0:1, :], axis=1, keepdims=True)
    d = jnp.sum(xw * a[1:2, :], axis=1, keepdims=True)
    return jnp.concatenate([s, d, jnp.zeros((xw.shape[0], 6), F32)], axis=1)


def _lin_first_body(x_ref, w_ref, a_ref, xw_ref, sd_ref):
    xw = jnp.dot(x_ref[...], w_ref[...], preferred_element_type=F32)
    xw_ref[...] = xw
    sd_ref[...] = _sd_from_xw(xw, a_ref[...])


def _lin_first(x, w, a2):
    N, DIN = x.shape
    return pl.pallas_call(
        _lin_first_body,
        grid=(N // _BN,),
        in_specs=[
            pl.BlockSpec((_BN, DIN), lambda i: (i, 0)),
            pl.BlockSpec((DIN, _D), lambda i: (0, 0)),
            pl.BlockSpec((2, _D), lambda i: (0, 0)),
        ],
        out_specs=[
            pl.BlockSpec((_BN, _D), lambda i: (i, 0)),
            pl.BlockSpec((_BN, 8), lambda i: (i, 0)),
        ],
        out_shape=[
            jax.ShapeDtypeStruct((N, _D), F32),
            jax.ShapeDtypeStruct((N, 8), F32),
        ],
    )(x, w, a2)


def _lin_mid_body(aa_ref, ab_ref, dp_ref, b_ref, w_ref, a_ref, xw_ref, sd_ref):
    den = jnp.sum(dp_ref[...], axis=1, keepdims=True) + 1e-16
    h = jnp.concatenate([aa_ref[...], ab_ref[...]], axis=1) / den
    h = jnp.maximum(h + b_ref[...], 0.0)
    xw = jnp.dot(h, w_ref[...], preferred_element_type=F32)
    xw_ref[...] = xw
    sd_ref[...] = _sd_from_xw(xw, a_ref[...])


def _lin_mid(acc_a, acc_b, dpt, b, w, a2):
    N = acc_a.shape[0]
    return pl.pallas_call(
        _lin_mid_body,
        grid=(N // _BN,),
        in_specs=[
            pl.BlockSpec((_BN, _DH), lambda i: (i, 0)),
            pl.BlockSpec((_BN, _DH), lambda i: (i, 0)),
            pl.BlockSpec((_BN, _NW), lambda i: (i, 0)),
            pl.BlockSpec((1, _D), lambda i: (0, 0)),
            pl.BlockSpec((_D, _D), lambda i: (0, 0)),
            pl.BlockSpec((2, _D), lambda i: (0, 0)),
        ],
        out_specs=[
            pl.BlockSpec((_BN, _D), lambda i: (i, 0)),
            pl.BlockSpec((_BN, 8), lambda i: (i, 0)),
        ],
        out_shape=[
            jax.ShapeDtypeStruct((N, _D), F32),
            jax.ShapeDtypeStruct((N, 8), F32),
        ],
    )(acc_a, acc_b, dpt, b, w, a2)


def _final_body(aa_ref, dp_ref, b_ref, out_ref):
    den = jnp.sum(dp_ref[...], axis=1, keepdims=True) + 1e-16
    out_ref[...] = aa_ref[...][:, 0:2] / den + b_ref[...][:, 0:2]


def _final(acc_a, dpt, b):
    N = acc_a.shape[0]
    return pl.pallas_call(
        _final_body,
        grid=(N // _BN,),
        in_specs=[
            pl.BlockSpec((_BN, _DH), lambda i: (i, 0)),
            pl.BlockSpec((_BN, _NW), lambda i: (i, 0)),
            pl.BlockSpec((1, 8), lambda i: (0, 0)),
        ],
        out_specs=pl.BlockSpec((_BN, 2), lambda i: (i, 0)),
        out_shape=jax.ShapeDtypeStruct((N, 2), F32),
    )(acc_a, dpt, b)


@functools.lru_cache(maxsize=None)
def _make_sc_alpha(N, E):
    per_w = E // _NW
    sup = 2000
    nsup = per_w // sup
    nch = sup // _C
    mesh = plsc.VectorSubcoreMesh(core_axis_name="c", subcore_axis_name="s")

    @functools.partial(
        pl.kernel,
        mesh=mesh,
        compiler_params=pltpu.CompilerParams(needs_layout_passes=False),
        out_type=[
            jax.ShapeDtypeStruct((E, _D), F32),
            jax.ShapeDtypeStruct((E,), F32),
            jax.ShapeDtypeStruct((_NW, N), F32),
        ],
        scratch_types=[
            pltpu.VMEM((N,), F32),
            pltpu.VMEM((N,), F32),
            pltpu.VMEM((N,), F32),
            pltpu.VMEM((3 * sup,), jnp.int32),
            pltpu.VMEM((sup,), F32),
            pltpu.VMEM((_C,), jnp.int32),
            pltpu.VMEM((_C, _D), F32),
            pltpu.SemaphoreType.DMA,
        ],
    )
    def sc_alpha(xw_hbm, s_hbm, d_hbm, ed_hbm, grows_hbm, a_hbm, dp_hbm,
                 s_loc, d_loc, den_loc, edw, abuf, idxbuf, rows, sem):
        cid = lax.axis_index("c")
        sid = lax.axis_index("s")
        wid = cid * _NT + sid
        base = wid * per_w
        pltpu.sync_copy(s_hbm, s_loc)
        pltpu.sync_copy(d_hbm, d_loc)
        zero16 = jnp.zeros((16,), F32)

        def zbody(i, carry):
            den_loc[pl.ds(i * 16, 16)] = zero16
            return carry

        lax.fori_loop(0, N // 16, zbody, 0)

        def sup_body(sc_i, carry2):
            sbase = pl.multiple_of(3 * (base + sc_i * sup), 8)
            pltpu.sync_copy(ed_hbm.at[pl.ds(sbase, 3 * sup)], edw)

            def chunk_body(ch, carry):
                ebase = ch * (3 * _C)
                for k in range(_C // 16):
                    g = ebase + k * 48
                    idxbuf[pl.ds(k * 16, 16)] = edw[pl.ds(g, 16)]
                pltpu.async_copy(xw_hbm.at[idxbuf], rows, sem).wait()
                gb = pl.multiple_of(base + sc_i * sup + ch * _C, 8)
                pltpu.sync_copy(rows, grows_hbm.at[pl.ds(gb, _C)])
                for k in range(_C // 16):
                    g = ebase + k * 48
                    i_s = edw[pl.ds(g, 16)]
                    i_d = edw[pl.ds(g + 16, 16)]
                    al = plsc.bitcast(edw[pl.ds(g + 32, 16)], F32)
                    al = al + plsc.load_gather(s_loc, [i_s])
                    al = al + plsc.load_gather(d_loc, [i_d])
                    al = jnp.maximum(al, al * 0.2)
                    a16 = jnp.exp(al)
                    abuf[pl.ds(ch * _C + k * 16, 16)] = a16
                    plsc.addupdate_scatter(den_loc, [i_d], a16)
                return carry

            lax.fori_loop(0, nch, chunk_body, 0)
            ab = pl.multiple_of(base + sc_i * sup, 8)
            pltpu.sync_copy(abuf, a_hbm.at[pl.ds(ab, sup)])
            return carry2

        lax.fori_loop(0, nsup, sup_body, 0)
        pltpu.sync_copy(den_loc, dp_hbm.at[wid])

    return sc_alpha


@functools.lru_cache(maxsize=None)
def _make_sc_scatter(N, E):
    per_t = E // _NT
    sup = 4000
    nsup = per_t // sup
    nch = sup // _C
    slab = 640
    slab_last = N - 15 * slab
    mesh = plsc.VectorSubcoreMesh(core_axis_name="c", subcore_axis_name="s")

    @functools.partial(
        pl.kernel,
        mesh=mesh,
        compiler_params=pltpu.CompilerParams(needs_layout_passes=False),
        out_type=jax.ShapeDtypeStruct((2 * N, _DH), F32),
        scratch_types=[
            pltpu.VMEM((3 * sup,), jnp.int32),
            pltpu.VMEM((_C,), jnp.int32),
            pltpu.VMEM((_C, _DH), F32),
            pltpu.VMEM_SHARED((N, _DH), F32),
            pltpu.SemaphoreType.DMA,
        ],
    )
    def sc_scatter(ma_hbm, mb_hbm, ed_hbm, zero_hbm, out_hbm,
                   edw, dstbuf, stag, acc_sh, sem):
        cid = lax.axis_index("c")
        sid = lax.axis_index("s")
        base = sid * per_t
        slab_off = pl.multiple_of(sid * slab, 8)

        @pl.when(sid < 15)
        def _():
            pltpu.sync_copy(zero_hbm.at[pl.ds(slab_off, slab)],
                            acc_sh.at[pl.ds(slab_off, slab)])

        @pl.when(sid == 15)
        def _():
            pltpu.sync_copy(zero_hbm.at[pl.ds(15 * slab, slab_last)],
                            acc_sh.at[pl.ds(15 * slab, slab_last)])

        plsc.subcore_barrier()

        def sup_body(sc_i, carry2):
            sbase = pl.multiple_of(3 * (base + sc_i * sup), 8)
            pltpu.sync_copy(ed_hbm.at[pl.ds(sbase, 3 * sup)], edw)

            def chunk_body(ch, carry):
                ebase = ch * (3 * _C)
                for k in range(_C // 16):
                    g = ebase + k * 48
                    dstbuf[pl.ds(k * 16, 16)] = edw[pl.ds(g + 16, 16)]
                mb = pl.multiple_of(base + sc_i * sup + ch * _C, 8)

                @pl.when(cid == 0)
                def _():
                    pltpu.sync_copy(ma_hbm.at[pl.ds(mb, _C)], stag)

                @pl.when(cid == 1)
                def _():
                    pltpu.sync_copy(mb_hbm.at[pl.ds(mb, _C)], stag)

                pltpu.sync_copy(stag, acc_sh.at[dstbuf], add=True)
                return carry

            lax.fori_loop(0, nch, chunk_body, 0)
            return carry2

        lax.fori_loop(0, nsup, sup_body, 0)
        plsc.subcore_barrier()
        out_off = pl.multiple_of(cid * N + sid * slab, 8)

        @pl.when(sid < 15)
        def _():
            pltpu.sync_copy(acc_sh.at[pl.ds(slab_off, slab)],
                            out_hbm.at[pl.ds(out_off, slab)])

        @pl.when(sid == 15)
        def _():
            pltpu.sync_copy(acc_sh.at[pl.ds(15 * slab, slab_last)],
                            out_hbm.at[pl.ds(cid * N + 15 * slab, slab_last)])

    return sc_scatter


def kernel(x, edge_index, edge_attr, params):
    N = x.shape[0]
    E = edge_index.shape[1]
    esrc = edge_index[0].reshape(E // 16, 1, 16)
    edst = edge_index[1].reshape(E // 16, 1, 16)

    ve = jnp.stack([p['We'] @ p['a_edge'] for p in params], axis=1)
    ea_all = _edge_feats(edge_attr, ve)

    zeros_acc = jnp.zeros((N, _DH), F32)
    sc_alpha = _make_sc_alpha(N, E)
    sc_scatter = _make_sc_scatter(N, E)

    a2 = jnp.stack([params[0]['a_src'], params[0]['a_dst']])
    xw, sd = _lin_first(x, params[0]['W'], a2)
    for l in range(8):
        eabits = ea_all[:, l].view(jnp.int32).reshape(E // 16, 1, 16)
        ed = jnp.concatenate([esrc, edst, eabits], axis=1).reshape(3 * E)
        grows, ae, dpart = sc_alpha(xw, sd[:, 0], sd[:, 1], ed)
        ma, mbh = _scale_rows(grows, ae[:, None])
        acc2 = sc_scatter(ma, mbh, ed, zeros_acc)
        acc_a, acc_b = acc2[:N], acc2[N:]
        dpt = dpart.T
        if l < 7:
            bl = params[l]['b'][None, :]
            wn = params[l + 1]['W']
            a2n = jnp.stack([params[l + 1]['a_src'], params[l + 1]['a_dst']])
            if l == 6:
                wn = jnp.pad(wn, ((0, 0), (0, _D - wn.shape[1])))
                a2n = jnp.pad(a2n, ((0, 0), (0, _D - a2n.shape[1])))
            xw, sd = _lin_mid(acc_a, acc_b, dpt, bl, wn, a2n)
    b7 = jnp.pad(params[7]['b'], (0, 6))[None, :]
    return _final(acc_a, dpt, b7)


# speedup vs baseline: 7.0471x; 1.0413x over previous
import functools

import jax
import jax.numpy as jnp
from jax import lax
from jax.experimental import pallas as pl
from jax.experimental.pallas import tpu as pltpu
from jax.experimental.pallas import tpu_sc as plsc

F32 = jnp.float32

_BN = 400
_BE = 2000
_D = 128
_DH = 64
_C = 80
_NW = 32
_NT = 16


def _edge_feat_body(ea_ref, ve_ref, out_ref):
    out_ref[...] = jnp.dot(ea_ref[...], ve_ref[...], preferred_element_type=F32)


def _edge_feats(edge_attr, ve):
    E, DE = edge_attr.shape
    return pl.pallas_call(
        _edge_feat_body,
        grid=(E // _BE,),
        in_specs=[
            pl.BlockSpec((_BE, DE), lambda i: (i, 0)),
            pl.BlockSpec(ve.shape, lambda i: (0, 0)),
        ],
        out_specs=pl.BlockSpec((_BE, ve.shape[1]), lambda i: (i, 0)),
        out_shape=jax.ShapeDtypeStruct((E, ve.shape[1]), F32),
    )(edge_attr, ve)


def _scale_body(g_ref, a_ref, ma_ref, mb_ref):
    m = g_ref[...] * a_ref[...]
    ma_ref[...] = m[:, :_DH]
    mb_ref[...] = m[:, _DH:]


def _scale_rows(grows, a2d):
    E = grows.shape[0]
    return pl.pallas_call(
        _scale_body,
        grid=(E // _BE,),
        in_specs=[
            pl.BlockSpec((_BE, _D), lambda i: (i, 0)),
            pl.BlockSpec((_BE, 1), lambda i: (i, 0)),
        ],
        out_specs=[
            pl.BlockSpec((_BE, _DH), lambda i: (i, 0)),
            pl.BlockSpec((_BE, _DH), lambda i: (i, 0)),
        ],
        out_shape=[
            jax.ShapeDtypeStruct((E, _DH), F32),
            jax.ShapeDtypeStruct((E, _DH), F32),
        ],
    )(grows, a2d)


def _sd_from_xw(xw, a):
    s = jnp.sum(xw * a[0:1, :], axis=1, keepdims=True)
    d = jnp.sum(xw * a[1:2, :], axis=1, keepdims=True)
    return jnp.concatenate([s, d, jnp.zeros((xw.shape[0], 6), F32)], axis=1)


def _lin_first_body(x_ref, w_ref, a_ref, xw_ref, sd_ref):
    xw = jnp.dot(x_ref[...], w_ref[...], preferred_element_type=F32)
    xw_ref[...] = xw
    sd_ref[...] = _sd_from_xw(xw, a_ref[...])


def _lin_first(x, w, a2):
    N, DIN = x.shape
    return pl.pallas_call(
        _lin_first_body,
        grid=(N // _BN,),
        in_specs=[
            pl.BlockSpec((_BN, DIN), lambda i: (i, 0)),
            pl.BlockSpec((DIN, _D), lambda i: (0, 0)),
            pl.BlockSpec((2, _D), lambda i: (0, 0)),
        ],
        out_specs=[
            pl.BlockSpec((_BN, _D), lambda i: (i, 0)),
            pl.BlockSpec((_BN, 8), lambda i: (i, 0)),
        ],
        out_shape=[
            jax.ShapeDtypeStruct((N, _D), F32),
            jax.ShapeDtypeStruct((N, 8), F32),
        ],
    )(x, w, a2)


def _lin_mid_body(aa_ref, ab_ref, dp_ref, b_ref, w_ref, a_ref, xw_ref, sd_ref):
    den = jnp.sum(dp_ref[...], axis=1, keepdims=True) + 1e-16
    h = jnp.concatenate([aa_ref[...], ab_ref[...]], axis=1) / den
    h = jnp.maximum(h + b_ref[...], 0.0)
    xw = jnp.dot(h, w_ref[...], preferred_element_type=F32)
    xw_ref[...] = xw
    sd_ref[...] = _sd_from_xw(xw, a_ref[...])


def _lin_mid(acc_a, acc_b, dpt, b, w, a2):
    N = acc_a.shape[0]
    return pl.pallas_call(
        _lin_mid_body,
        grid=(N // _BN,),
        in_specs=[
            pl.BlockSpec((_BN, _DH), lambda i: (i, 0)),
            pl.BlockSpec((_BN, _DH), lambda i: (i, 0)),
            pl.BlockSpec((_BN, _NW), lambda i: (i, 0)),
            pl.BlockSpec((1, _D), lambda i: (0, 0)),
            pl.BlockSpec((_D, _D), lambda i: (0, 0)),
            pl.BlockSpec((2, _D), lambda i: (0, 0)),
        ],
        out_specs=[
            pl.BlockSpec((_BN, _D), lambda i: (i, 0)),
            pl.BlockSpec((_BN, 8), lambda i: (i, 0)),
        ],
        out_shape=[
            jax.ShapeDtypeStruct((N, _D), F32),
            jax.ShapeDtypeStruct((N, 8), F32),
        ],
    )(acc_a, acc_b, dpt, b, w, a2)


def _final_body(aa_ref, dp_ref, b_ref, out_ref):
    den = jnp.sum(dp_ref[...], axis=1, keepdims=True) + 1e-16
    out_ref[...] = aa_ref[...][:, 0:2] / den + b_ref[...][:, 0:2]


def _final(acc_a, dpt, b):
    N = acc_a.shape[0]
    return pl.pallas_call(
        _final_body,
        grid=(N // _BN,),
        in_specs=[
            pl.BlockSpec((_BN, _DH), lambda i: (i, 0)),
            pl.BlockSpec((_BN, _NW), lambda i: (i, 0)),
            pl.BlockSpec((1, 8), lambda i: (0, 0)),
        ],
        out_specs=pl.BlockSpec((_BN, 2), lambda i: (i, 0)),
        out_shape=jax.ShapeDtypeStruct((N, 2), F32),
    )(acc_a, dpt, b)


@functools.lru_cache(maxsize=None)
def _make_sc_alpha(N, E):
    per_w = E // _NW
    sup = 2000
    nsup = per_w // sup
    nch = sup // _C
    mesh = plsc.VectorSubcoreMesh(core_axis_name="c", subcore_axis_name="s")

    @functools.partial(
        pl.kernel,
        mesh=mesh,
        compiler_params=pltpu.CompilerParams(needs_layout_passes=False),
        out_type=[
            jax.ShapeDtypeStruct((E, _D), F32),
            jax.ShapeDtypeStruct((E,), F32),
            jax.ShapeDtypeStruct((_NW, N), F32),
        ],
        scratch_types=[
            pltpu.VMEM((N,), F32),
            pltpu.VMEM((N,), F32),
            pltpu.VMEM((N,), F32),
            pltpu.VMEM((3 * sup,), jnp.int32),
            pltpu.VMEM((sup,), F32),
            pltpu.VMEM((_C,), jnp.int32),
            pltpu.VMEM((_C,), jnp.int32),
            pltpu.VMEM((_C, _D), F32),
            pltpu.VMEM((_C, _D), F32),
            pltpu.SemaphoreType.DMA,
        ],
    )
    def sc_alpha(xw_hbm, s_hbm, d_hbm, ed_hbm, grows_hbm, a_hbm, dp_hbm,
                 s_loc, d_loc, den_loc, edw, abuf, idxbuf0, idxbuf1, rows0,
                 rows1, sem):
        cid = lax.axis_index("c")
        sid = lax.axis_index("s")
        wid = cid * _NT + sid
        base = wid * per_w
        pltpu.sync_copy(s_hbm, s_loc)
        pltpu.sync_copy(d_hbm, d_loc)
        zero16 = jnp.zeros((16,), F32)

        def zbody(i, carry):
            den_loc[pl.ds(i * 16, 16)] = zero16
            return carry

        lax.fori_loop(0, N // 16, zbody, 0)

        def build_idx(ch, ib):
            for k in range(_C // 16):
                g = ch * (3 * _C) + k * 48
                ib[pl.ds(k * 16, 16)] = edw[pl.ds(g, 16)]

        def sup_body(sc_i, carry2):
            sbase = pl.multiple_of(3 * (base + sc_i * sup), 8)
            pltpu.sync_copy(ed_hbm.at[pl.ds(sbase, 3 * sup)], edw)
            build_idx(0, idxbuf0)
            pltpu.async_copy(xw_hbm.at[idxbuf0], rows0, sem)

            def chunk_body(ch, carry):
                ebase = ch * (3 * _C)
                gb = pl.multiple_of(base + sc_i * sup + ch * _C, 8)

                @pl.when(ch % 2 == 0)
                def _():
                    pltpu.make_async_copy(xw_hbm.at[idxbuf0], rows0,
                                          sem).wait()

                    @pl.when(ch + 1 < nch)
                    def _():
                        build_idx(ch + 1, idxbuf1)
                        pltpu.async_copy(xw_hbm.at[idxbuf1], rows1, sem)

                    pltpu.sync_copy(rows0, grows_hbm.at[pl.ds(gb, _C)])

                @pl.when(ch % 2 == 1)
                def _():
                    pltpu.make_async_copy(xw_hbm.at[idxbuf1], rows1,
                                          sem).wait()

                    @pl.when(ch + 1 < nch)
                    def _():
                        build_idx(ch + 1, idxbuf0)
                        pltpu.async_copy(xw_hbm.at[idxbuf0], rows0, sem)

                    pltpu.sync_copy(rows1, grows_hbm.at[pl.ds(gb, _C)])

                for k in range(_C // 16):
                    g = ebase + k * 48
                    i_s = edw[pl.ds(g, 16)]
                    i_d = edw[pl.ds(g + 16, 16)]
                    al = plsc.bitcast(edw[pl.ds(g + 32, 16)], F32)
                    al = al + plsc.load_gather(s_loc, [i_s])
                    al = al + plsc.load_gather(d_loc, [i_d])
                    al = jnp.maximum(al, al * 0.2)
                    a16 = jnp.exp(al)
                    abuf[pl.ds(ch * _C + k * 16, 16)] = a16
                    plsc.addupdate_scatter(den_loc, [i_d], a16)
                return carry

            lax.fori_loop(0, nch, chunk_body, 0)
            ab = pl.multiple_of(base + sc_i * sup, 8)
            pltpu.sync_copy(abuf, a_hbm.at[pl.ds(ab, sup)])
            return carry2

        lax.fori_loop(0, nsup, sup_body, 0)
        pltpu.sync_copy(den_loc, dp_hbm.at[wid])

    return sc_alpha


@functools.lru_cache(maxsize=None)
def _make_sc_scatter(N, E):
    per_t = E // _NT
    sup = 4000
    nsup = per_t // sup
    nch = sup // _C
    slab = 640
    slab_last = N - 15 * slab
    mesh = plsc.VectorSubcoreMesh(core_axis_name="c", subcore_axis_name="s")

    @functools.partial(
        pl.kernel,
        mesh=mesh,
        compiler_params=pltpu.CompilerParams(needs_layout_passes=False),
        out_type=jax.ShapeDtypeStruct((2 * N, _DH), F32),
        scratch_types=[
            pltpu.VMEM((3 * sup,), jnp.int32),
            pltpu.VMEM((_C,), jnp.int32),
            pltpu.VMEM((_C, _DH), F32),
            pltpu.VMEM_SHARED((N, _DH), F32),
            pltpu.SemaphoreType.DMA,
        ],
    )
    def sc_scatter(ma_hbm, mb_hbm, ed_hbm, zero_hbm, out_hbm,
                   edw, dstbuf, stag, acc_sh, sem):
        cid = lax.axis_index("c")
        sid = lax.axis_index("s")
        base = sid * per_t
        slab_off = pl.multiple_of(sid * slab, 8)

        @pl.when(sid < 15)
        def _():
            pltpu.sync_copy(zero_hbm.at[pl.ds(slab_off, slab)],
                            acc_sh.at[pl.ds(slab_off, slab)])

        @pl.when(sid == 15)
        def _():
            pltpu.sync_copy(zero_hbm.at[pl.ds(15 * slab, slab_last)],
                            acc_sh.at[pl.ds(15 * slab, slab_last)])

        plsc.subcore_barrier()

        def sup_body(sc_i, carry2):
            sbase = pl.multiple_of(3 * (base + sc_i * sup), 8)
            pltpu.sync_copy(ed_hbm.at[pl.ds(sbase, 3 * sup)], edw)

            def chunk_body(ch, carry):
                ebase = ch * (3 * _C)
                for k in range(_C // 16):
                    g = ebase + k * 48
                    dstbuf[pl.ds(k * 16, 16)] = edw[pl.ds(g + 16, 16)]
                mb = pl.multiple_of(base + sc_i * sup + ch * _C, 8)

                @pl.when(cid == 0)
                def _():
                    pltpu.sync_copy(ma_hbm.at[pl.ds(mb, _C)], stag)

                @pl.when(cid == 1)
                def _():
                    pltpu.sync_copy(mb_hbm.at[pl.ds(mb, _C)], stag)

                pltpu.sync_copy(stag, acc_sh.at[dstbuf], add=True)
                return carry

            lax.fori_loop(0, nch, chunk_body, 0)
            return carry2

        lax.fori_loop(0, nsup, sup_body, 0)
        plsc.subcore_barrier()
        out_off = pl.multiple_of(cid * N + sid * slab, 8)

        @pl.when(sid < 15)
        def _():
            pltpu.sync_copy(acc_sh.at[pl.ds(slab_off, slab)],
                            out_hbm.at[pl.ds(out_off, slab)])

        @pl.when(sid == 15)
        def _():
            pltpu.sync_copy(acc_sh.at[pl.ds(15 * slab, slab_last)],
                            out_hbm.at[pl.ds(cid * N + 15 * slab, slab_last)])

    return sc_scatter


def kernel(x, edge_index, edge_attr, params):
    N = x.shape[0]
    E = edge_index.shape[1]
    esrc = edge_index[0].reshape(E // 16, 1, 16)
    edst = edge_index[1].reshape(E // 16, 1, 16)

    ve = jnp.stack([p['We'] @ p['a_edge'] for p in params], axis=1)
    ea_all = _edge_feats(edge_attr, ve)

    zeros_acc = jnp.zeros((N, _DH), F32)
    sc_alpha = _make_sc_alpha(N, E)
    sc_scatter = _make_sc_scatter(N, E)

    a2 = jnp.stack([params[0]['a_src'], params[0]['a_dst']])
    xw, sd = _lin_first(x, params[0]['W'], a2)
    for l in range(8):
        eabits = ea_all[:, l].view(jnp.int32).reshape(E // 16, 1, 16)
        ed = jnp.concatenate([esrc, edst, eabits], axis=1).reshape(3 * E)
        grows, ae, dpart = sc_alpha(xw, sd[:, 0], sd[:, 1], ed)
        ma, mbh = _scale_rows(grows, ae[:, None])
        acc2 = sc_scatter(ma, mbh, ed, zeros_acc)
        acc_a, acc_b = acc2[:N], acc2[N:]
        dpt = dpart.T
        if l < 7:
            bl = params[l]['b'][None, :]
            wn = params[l + 1]['W']
            a2n = jnp.stack([params[l + 1]['a_src'], params[l + 1]['a_dst']])
            if l == 6:
                wn = jnp.pad(wn, ((0, 0), (0, _D - wn.shape[1])))
                a2n = jnp.pad(a2n, ((0, 0), (0, _D - a2n.shape[1])))
            xw, sd = _lin_mid(acc_a, acc_b, dpt, bl, wn, a2n)
    b7 = jnp.pad(params[7]['b'], (0, 6))[None, :]
    return _final(acc_a, dpt, b7)
